# baseline (speedup 1.0000x reference)
import functools

import jax
import jax.numpy as jnp
from jax import lax
from jax.experimental import pallas as pl
from jax.experimental.pallas import tpu as pltpu
from jax.experimental.pallas import tpu_sc as plsc

N_NODES = 10000
D = 128
E = 320000
LOSS_WEIGHT = 0.1

NC = 2
NS = 16
NW = NC * NS
L = 16

NPAD = 10240
RPT = NPAD // NS
EPW = E // NW
K = 80
NCH = EPW // K
NPAIR = (NCH - 1) // 2

_mesh = plsc.VectorSubcoreMesh(core_axis_name="c", subcore_axis_name="s")


@functools.partial(
    pl.kernel,
    out_type=[
        jax.ShapeDtypeStruct((NC, NPAD, D), jnp.bfloat16),
        jax.ShapeDtypeStruct((NC, NPAD, L), jnp.int32),
    ],
    mesh=_mesh,
    compiler_params=pltpu.CompilerParams(use_tc_tiling_on_sc=False),
    scratch_types=[
        pltpu.VMEM((EPW,), jnp.int32),
        pltpu.VMEM((EPW,), jnp.int32),
        pltpu.VMEM((K, D), jnp.bfloat16),
        pltpu.VMEM((K, D), jnp.bfloat16),
        pltpu.VMEM((K, L), jnp.int32),
        pltpu.VMEM_SHARED((NPAD, D), jnp.bfloat16),
        pltpu.VMEM_SHARED((NPAD, L), jnp.int32),
        pltpu.SemaphoreType.DMA,
        pltpu.SemaphoreType.DMA,
        pltpu.SemaphoreType.DMA,
        pltpu.SemaphoreType.DMA,
    ],
)
def _edge_scatter(fb_hbm, edges_hbm, zf_hbm, zi_hbm, s_out, cnt_out,
                  idx_r, idx_c, cbuf0, cbuf1, ones_buf, s_acc, cnt_acc,
                  sem_g0, sem_s0, sem_g1, sem_s1):
    cid = lax.axis_index("c")
    sid = lax.axis_index("s")
    wid = sid * NC + cid
    base = wid * EPW
    srow = sid * RPT

    pltpu.sync_copy(zf_hbm, s_acc.at[pl.ds(srow, RPT)])
    pltpu.sync_copy(zi_hbm, cnt_acc.at[pl.ds(srow, RPT)])
    pltpu.sync_copy(edges_hbm.at[pl.ds(base, EPW)], idx_r)
    pltpu.sync_copy(edges_hbm.at[pl.ds(E + base, EPW)], idx_c)

    ones16 = jnp.ones((L,), jnp.int32)

    def ones_body(e, _):
        ones_buf[e, :] = ones16
        return 0

    lax.fori_loop(0, K, ones_body, 0)

    cbufs = (cbuf0, cbuf1)
    sems_g = (sem_g0, sem_g1)
    sems_s = (sem_s0, sem_s1)

    def start_g(ci, b):
        pltpu.async_copy(
            fb_hbm.at[idx_c.at[pl.ds(ci * K, K)]], cbufs[b], sems_g[b])

    def wait_g(ci, b):
        pltpu.make_async_copy(
            fb_hbm.at[idx_c.at[pl.ds(ci * K, K)]], cbufs[b],
            sems_g[b]).wait()

    def start_s(ci, b):
        pltpu.async_copy(
            cbufs[b], s_acc.at[idx_r.at[pl.ds(ci * K, K)]], sems_s[b],
            add=True)
        pltpu.async_copy(
            ones_buf, cnt_acc.at[idx_r.at[pl.ds(ci * K, K)]], sems_s[b],
            add=True)
        pltpu.async_copy(
            ones_buf, cnt_acc.at[idx_c.at[pl.ds(ci * K, K)]], sems_s[b],
            add=True)

    def wait_s(ci, b):
        pltpu.make_async_copy(
            cbufs[b], s_acc.at[idx_r.at[pl.ds(ci * K, K)]],
            sems_s[b]).wait()
        pltpu.make_async_copy(
            ones_buf, cnt_acc.at[idx_r.at[pl.ds(ci * K, K)]],
            sems_s[b]).wait()
        pltpu.make_async_copy(
            ones_buf, cnt_acc.at[idx_c.at[pl.ds(ci * K, K)]],
            sems_s[b]).wait()

    start_g(0, 0)
    start_g(1, 1)

    plsc.subcore_barrier()

    def pair_body(i, _):
        c0 = 2 * i
        c1 = c0 + 1
        wait_g(c0, 0)
        start_s(c0, 0)
        wait_g(c1, 1)
        start_s(c1, 1)
        wait_s(c0, 0)
        start_g(c0 + 2, 0)
        wait_s(c1, 1)

        @pl.when(c1 + 2 < NCH)
        def _():
            start_g(c1 + 2, 1)

        return 0

    lax.fori_loop(0, NPAIR, pair_body, 0)

    wait_g(NCH - 1, 0)
    start_s(NCH - 1, 0)
    wait_s(NCH - 1, 0)

    plsc.subcore_barrier()
    pltpu.sync_copy(s_acc.at[pl.ds(srow, RPT)],
                    s_out.at[cid, pl.ds(srow, RPT)])
    pltpu.sync_copy(cnt_acc.at[pl.ds(srow, RPT)],
                    cnt_out.at[cid, pl.ds(srow, RPT)])


_BLK = 512
_NBLK = NPAD // _BLK


def _combine_body(f_ref, c_ref, s_ref, out_ref):
    g = pl.program_id(0)

    @pl.when(g == 0)
    def _():
        out_ref[0, 0] = 0.0

    f = f_ref[...]
    deg16 = jnp.sum(c_ref[...], axis=(0, 2)).astype(jnp.float32)
    nsq = jnp.sum(f * f, axis=1)
    t1 = jnp.sum(deg16 * nsq) * (1.0 / L)
    s = s_ref[0].astype(jnp.float32) + s_ref[1].astype(jnp.float32)
    t3 = jnp.sum(f * s)
    out_ref[0, 0] += t1 - 2.0 * t3


_dense_combine = pl.pallas_call(
    _combine_body,
    grid=(_NBLK,),
    in_specs=[
        pl.BlockSpec((_BLK, D), lambda g: (g, 0)),
        pl.BlockSpec((NC, _BLK, L), lambda g: (0, g, 0)),
        pl.BlockSpec((NC, _BLK, D), lambda g: (0, g, 0)),
    ],
    out_specs=pl.BlockSpec(memory_space=pltpu.SMEM),
    out_shape=jax.ShapeDtypeStruct((1, 1), jnp.float32),
)


def kernel(features, edge_index):
    fb = features.astype(jnp.bfloat16)
    zf = jnp.zeros((RPT, D), jnp.bfloat16)
    zi = jnp.zeros((RPT, L), jnp.int32)
    s2, cnt2 = _edge_scatter(fb, edge_index.reshape(2 * E), zf, zi)
    fpad = jnp.pad(features, ((0, NPAD - N_NODES), (0, 0)))
    total = _dense_combine(fpad, cnt2, s2)
    return (LOSS_WEIGHT / (E * D)) * total[0, 0]


# speedup vs baseline: 1.1837x; 1.1837x over previous
import functools

import jax
import jax.numpy as jnp
from jax import lax
from jax.experimental import pallas as pl
from jax.experimental.pallas import tpu as pltpu
from jax.experimental.pallas import tpu_sc as plsc

N_NODES = 10000
D = 128
E = 320000
LOSS_WEIGHT = 0.1

NC = 2
NS = 16
NW = NC * NS
L = 16

W = D // 2
EPW = E // NW
K = 80
NCH = EPW // K
NPAIR = (NCH - 1) // 2

_mesh = plsc.VectorSubcoreMesh(core_axis_name="c", subcore_axis_name="s")


@functools.partial(
    pl.kernel,
    out_type=jax.ShapeDtypeStruct((NW, L), jnp.float32),
    mesh=_mesh,
    compiler_params=pltpu.CompilerParams(use_tc_tiling_on_sc=False),
    scratch_types=[
        pltpu.VMEM((EPW,), jnp.int32),
        pltpu.VMEM((EPW,), jnp.int32),
        pltpu.VMEM((K, W), jnp.int32),
        pltpu.VMEM((K, W), jnp.int32),
        pltpu.VMEM((K, W), jnp.int32),
        pltpu.VMEM((K, W), jnp.int32),
        pltpu.VMEM((L,), jnp.float32),
        pltpu.SemaphoreType.DMA,
        pltpu.SemaphoreType.DMA,
        pltpu.SemaphoreType.DMA,
        pltpu.SemaphoreType.DMA,
    ],
)
def _edge_sq_sum(fpacked_hbm, edges_hbm, out_hbm,
                 idx_r, idx_c, rbuf0, cbuf0, rbuf1, cbuf1, acc_v,
                 sem_r0, sem_c0, sem_r1, sem_c1):
    wid = lax.axis_index("s") * NC + lax.axis_index("c")
    base = wid * EPW

    pltpu.sync_copy(edges_hbm.at[pl.ds(base, EPW)], idx_r)
    pltpu.sync_copy(edges_hbm.at[pl.ds(E + base, EPW)], idx_c)

    rbufs = (rbuf0, rbuf1)
    cbufs = (cbuf0, cbuf1)
    sems_r = (sem_r0, sem_r1)
    sems_c = (sem_c0, sem_c1)

    def start_rc(ci, b):
        pltpu.async_copy(
            fpacked_hbm.at[idx_r.at[pl.ds(ci * K, K)]], rbufs[b], sems_r[b])
        pltpu.async_copy(
            fpacked_hbm.at[idx_c.at[pl.ds(ci * K, K)]], cbufs[b], sems_c[b])

    def wait_rc(ci, b):
        pltpu.make_async_copy(
            fpacked_hbm.at[idx_r.at[pl.ds(ci * K, K)]], rbufs[b],
            sems_r[b]).wait()
        pltpu.make_async_copy(
            fpacked_hbm.at[idx_c.at[pl.ds(ci * K, K)]], cbufs[b],
            sems_c[b]).wait()

    nacc = 2 * (W // L)
    shift = jnp.full((L,), 16, jnp.int32)
    hi_mask = jnp.full((L,), -65536, jnp.int32)

    def chunk_sum(b, accs):
        def edge_body(e, accs):
            out = list(accs)
            for g in range(W // L):
                wr = rbufs[b][e, pl.ds(g * L, L)]
                wc = cbufs[b][e, pl.ds(g * L, L)]
                rl = lax.bitcast_convert_type(lax.shift_left(wr, shift), jnp.float32)
                cl = lax.bitcast_convert_type(lax.shift_left(wc, shift), jnp.float32)
                dl = rl - cl
                out[2 * g] = out[2 * g] + dl * dl
                rh = lax.bitcast_convert_type(lax.bitwise_and(wr, hi_mask), jnp.float32)
                ch = lax.bitcast_convert_type(lax.bitwise_and(wc, hi_mask), jnp.float32)
                dh = rh - ch
                out[2 * g + 1] = out[2 * g + 1] + dh * dh
            return tuple(out)
        return plsc.parallel_loop(0, K, 1, unroll=2, carry=accs)(edge_body)

    start_rc(0, 0)
    start_rc(1, 1)

    def pair_body(i, accs):
        c0 = 2 * i
        c1 = c0 + 1
        wait_rc(c0, 0)
        accs = chunk_sum(0, accs)
        start_rc(c0 + 2, 0)
        wait_rc(c1, 1)
        accs = chunk_sum(1, accs)

        @pl.when(c1 + 2 < NCH)
        def _():
            start_rc(c1 + 2, 1)

        return accs

    zeros = tuple(jnp.zeros((L,), jnp.float32) for _ in range(nacc))
    accs = lax.fori_loop(0, NPAIR, pair_body, zeros)

    wait_rc(NCH - 1, 0)
    accs = chunk_sum(0, accs)

    acc = ((accs[0] + accs[1]) + (accs[2] + accs[3])) + (
        (accs[4] + accs[5]) + (accs[6] + accs[7]))
    acc_v[...] = acc
    pltpu.sync_copy(acc_v, out_hbm.at[wid])


def kernel(features, edge_index):
    fb = features.astype(jnp.bfloat16)
    fpacked = lax.bitcast_convert_type(
        fb.reshape(N_NODES, W, 2), jnp.int32)
    partials = _edge_sq_sum(fpacked, edge_index.reshape(2 * E))
    return (LOSS_WEIGHT / (E * D)) * jnp.sum(partials)
